# baseline (speedup 1.0000x reference)
import functools
import math

import numpy as np

import jax
import jax.numpy as jnp
from jax import lax
from jax.experimental import pallas as pl
from jax.experimental.pallas import tpu as pltpu
from jax.experimental.pallas import tpu_sc as plsc

N = 10000
E = 320000
D_STR = 58
ALPHA = 0.4
THETA = 0.9

N_PAD = 10240
NTILES = 32
CHUNK = 128
G_C0 = 104
G_C1 = 56
GMAX = max(G_C0, G_C1)
E_CAP = 16 * (G_C0 + G_C1) * CHUNK
ROWS_PER_TILE = N_PAD // 16

_MESH = plsc.VectorSubcoreMesh(core_axis_name="c", subcore_axis_name="s")

_E_SPLIT = 16 * G_C0 * CHUNK


def _edge_layout(flat, pad_value):
    a = flat[:_E_SPLIT].reshape(16, G_C0, CHUNK)
    b = flat[_E_SPLIT:].reshape(16, G_C1, CHUNK)
    if G_C0 > G_C1:
        b = jnp.pad(b, ((0, 0), (0, GMAX - G_C1), (0, 0)), constant_values=pad_value)
    else:
        a = jnp.pad(a, ((0, 0), (0, GMAX - G_C0), (0, 0)), constant_values=pad_value)
    return jnp.concatenate([a, b], axis=0)


@functools.partial(
    pl.kernel,
    out_type=jax.ShapeDtypeStruct((NTILES, N_PAD), jnp.float32),
    mesh=_MESH,
    compiler_params=pltpu.CompilerParams(needs_layout_passes=False),
    scratch_types=[
        pltpu.VMEM((GMAX, CHUNK), jnp.int32),
        pltpu.VMEM((N_PAD,), jnp.float32),
    ],
)
def _deg_kernel(col_hbm, zeros1d_hbm, out_hbm, col_vm, acc_vm):
    c = lax.axis_index("c")
    s = lax.axis_index("s")
    w = c * 16 + s
    pltpu.sync_copy(zeros1d_hbm, acc_vm)
    pltpu.sync_copy(col_hbm.at[w], col_vm)
    ones = jnp.full((16,), 1.0, dtype=jnp.float32)

    def body(g, carry):
        for j in range(CHUNK // 16):
            idx = col_vm[g, pl.ds(j * 16, 16)]
            plsc.addupdate_scatter(acc_vm, [idx], ones)
        return carry

    @pl.when(c == 0)
    def _():
        lax.fori_loop(0, G_C0, body, 0)

    @pl.when(c != 0)
    def _():
        lax.fori_loop(0, G_C1, body, 0)

    pltpu.sync_copy(acc_vm, out_hbm.at[w])


@functools.partial(
    pl.kernel,
    out_type=jax.ShapeDtypeStruct((2, N_PAD, 128), jnp.float32),
    mesh=_MESH,
    scratch_types=[
        pltpu.VMEM((GMAX, CHUNK), jnp.int32),
        pltpu.VMEM((GMAX, CHUNK), jnp.int32),
        pltpu.VMEM((CHUNK, 128), jnp.float32),
        pltpu.SemaphoreType.DMA,
        pltpu.VMEM_SHARED((N_PAD, 128), jnp.float32),
    ],
)
def _msg_kernel(hs_hbm, row_hbm, col_hbm, zeros2d_hbm, out_hbm,
                row_vm, col_vm, buf, sem, acc_sh):
    c = lax.axis_index("c")
    s = lax.axis_index("s")
    w = c * 16 + s
    pltpu.sync_copy(zeros2d_hbm, acc_sh.at[pl.ds(s * ROWS_PER_TILE, ROWS_PER_TILE)])
    pltpu.sync_copy(row_hbm.at[w], row_vm)
    pltpu.sync_copy(col_hbm.at[w], col_vm)
    plsc.subcore_barrier()

    def body(g, carry):
        pltpu.async_copy(hs_hbm.at[row_vm.at[g]], buf, sem).wait()
        pltpu.sync_copy(buf, acc_sh.at[col_vm.at[g]], add=True)
        return carry

    @pl.when(c == 0)
    def _():
        lax.fori_loop(0, G_C0, body, 0)

    @pl.when(c != 0)
    def _():
        lax.fori_loop(0, G_C1, body, 0)

    plsc.subcore_barrier()
    pltpu.sync_copy(
        acc_sh.at[pl.ds(s * ROWS_PER_TILE, ROWS_PER_TILE)],
        out_hbm.at[c, pl.ds(s * ROWS_PER_TILE, ROWS_PER_TILE)],
    )


_BLK = 2048
_GRID = N_PAD // _BLK
_DOT = dict(preferred_element_type=jnp.float32, precision=jax.lax.Precision.HIGHEST)


def _pre_body(xcat_ref, degp_ref, wblk_ref, bcat_ref, h0_ref, hs_ref, dis_ref):
    deg = jnp.sum(degp_ref[...], axis=0)
    dis = jnp.where(deg > 0, lax.rsqrt(jnp.maximum(deg, 1e-12)), 0.0)
    h = jnp.maximum(jnp.dot(xcat_ref[...], wblk_ref[...], **_DOT) + bcat_ref[...], 0.0)
    h0_ref[...] = h
    hs_ref[...] = h * dis[:, None]
    dis_ref[...] = jnp.broadcast_to(dis[:, None], (_BLK, 128))


_pre_call = pl.pallas_call(
    _pre_body,
    grid=(_GRID,),
    in_specs=[
        pl.BlockSpec((_BLK, 192), lambda i: (i, 0)),
        pl.BlockSpec((NTILES, _BLK), lambda i: (0, i)),
        pl.BlockSpec((192, 128), lambda i: (0, 0)),
        pl.BlockSpec((1, 128), lambda i: (0, 0)),
    ],
    out_specs=[
        pl.BlockSpec((_BLK, 128), lambda i: (i, 0)),
        pl.BlockSpec((_BLK, 128), lambda i: (i, 0)),
        pl.BlockSpec((_BLK, 128), lambda i: (i, 0)),
    ],
    out_shape=[jax.ShapeDtypeStruct((N_PAD, 128), jnp.float32)] * 3,
)


def _layer_body(agg_ref, h0_ref, dis_ref, w1_ref, w2_ref, out_ref, *, beta):
    agg = (agg_ref[0] + agg_ref[1]) * dis_ref[...] * (1.0 - ALPHA)
    h0a = h0_ref[...] * ALPHA
    t = (1.0 - beta) * (agg + h0a) + beta * (
        jnp.dot(agg, w1_ref[...], **_DOT) + jnp.dot(h0a, w2_ref[...], **_DOT))
    out_ref[...] = jnp.maximum(t, 0.0) * dis_ref[...]


def _final_body(agg_ref, h0_ref, dis_ref, w1_ref, w2_ref, wz_ref, bz_ref,
                out_ref, *, beta):
    agg = (agg_ref[0] + agg_ref[1]) * dis_ref[...] * (1.0 - ALPHA)
    h0a = h0_ref[...] * ALPHA
    t = (1.0 - beta) * (agg + h0a) + beta * (
        jnp.dot(agg, w1_ref[...], **_DOT) + jnp.dot(h0a, w2_ref[...], **_DOT))
    h = jnp.maximum(t, 0.0)
    out_ref[...] = jnp.dot(h, wz_ref[...], **_DOT) + bz_ref[...]


def _make_layer_call(body, beta, extra_specs=(), n_extra=0):
    return pl.pallas_call(
        functools.partial(body, beta=beta),
        grid=(_GRID,),
        in_specs=[
            pl.BlockSpec((2, _BLK, 128), lambda i: (0, i, 0)),
            pl.BlockSpec((_BLK, 128), lambda i: (i, 0)),
            pl.BlockSpec((_BLK, 128), lambda i: (i, 0)),
            pl.BlockSpec((128, 128), lambda i: (0, 0)),
            pl.BlockSpec((128, 128), lambda i: (0, 0)),
            *extra_specs,
        ],
        out_specs=pl.BlockSpec((_BLK, 128), lambda i: (i, 0)),
        out_shape=jax.ShapeDtypeStruct((N_PAD, 128), jnp.float32),
    )


def _beta(i):
    return float(math.log(THETA / (i + 1) + 1.0))


_layer1_call = _make_layer_call(_layer_body, _beta(0))
_final_call = _make_layer_call(
    _final_body, _beta(1),
    extra_specs=(pl.BlockSpec((128, 128), lambda i: (0, 0)),
                 pl.BlockSpec((1, 128), lambda i: (0, 0))),
)


def kernel(x, data_str, edge_index, lins0_w, lins0_b, lins1_w, lins1_b,
           lin11_w, lin11_b, lin3_w, lin3_b,
           convs_w1, convs_w2, convs1_w1, convs1_w2):
    f32 = jnp.float32
    row = edge_index[0]
    col = edge_index[1]
    row_p = jnp.concatenate([row, jnp.zeros((E_CAP - E,), jnp.int32)])
    col_p = jnp.concatenate([col, jnp.full((E_CAP - E,), N, jnp.int32)])
    row_t = _edge_layout(row_p, 0)
    col_t = _edge_layout(col_p, N)

    x_p = jnp.pad(x, ((0, N_PAD - N), (0, 0)))
    ds_p = jnp.pad(data_str, ((0, N_PAD - N), (0, 64 - D_STR)))
    xcat = jnp.concatenate([x_p, ds_p], axis=1)

    zero64 = jnp.zeros((64, 64), f32)
    wblk = jnp.concatenate([
        jnp.concatenate([lins0_w, jnp.zeros((128, 64), f32)], axis=1),
        jnp.concatenate([jnp.zeros((64, 64), f32),
                         jnp.pad(lin11_w, ((0, 64 - D_STR), (0, 0)))], axis=1),
    ], axis=0)
    bcat = jnp.concatenate([lins0_b, lin11_b]).reshape(1, 128)

    def blkdiag(a, b):
        return jnp.concatenate([
            jnp.concatenate([a, zero64], axis=1),
            jnp.concatenate([zero64, b], axis=1),
        ], axis=0)

    w1b = [blkdiag(convs_w1[i], convs1_w1[i]) for i in range(2)]
    w2b = [blkdiag(convs_w2[i], convs1_w2[i]) for i in range(2)]

    wz = jnp.zeros((128, 128), f32)
    wz = wz.at[:64, 0:1].set(lins1_w)
    wz = wz.at[64:, 1:2].set(lin3_w)
    bz = jnp.zeros((1, 128), f32)
    bz = bz.at[0, 0].set(lins1_b[0])
    bz = bz.at[0, 1].set(lin3_b[0])

    zeros1d = jnp.zeros((N_PAD,), f32)
    zeros2d = jnp.zeros((ROWS_PER_TILE, 128), f32)

    degp = _deg_kernel(col_t, zeros1d)
    h0, hs, dis = _pre_call(xcat, degp, wblk, bcat)

    agg1 = _msg_kernel(hs, row_t, col_t, zeros2d)
    hs1 = _layer1_call(agg1, h0, dis, w1b[0], w2b[0])

    agg2 = _msg_kernel(hs1, row_t, col_t, zeros2d)
    zcat = _final_call(agg2, h0, dis, w1b[1], w2b[1], wz, bz)

    return (zcat[:N, 0:1], zcat[:N, 1:2])


# speedup vs baseline: 1.4675x; 1.4675x over previous
import functools
import math

import numpy as np

import jax
import jax.numpy as jnp
from jax import lax
from jax.experimental import pallas as pl
from jax.experimental.pallas import tpu as pltpu
from jax.experimental.pallas import tpu_sc as plsc

N = 10000
E = 320000
D_STR = 58
ALPHA = 0.4
THETA = 0.9

N_PAD = 10240
NTILES = 32
CHUNK = 128
G_C0 = 108
G_C1 = 50
GMAX = max(G_C0, G_C1)
E_CAP = 16 * (G_C0 + G_C1) * CHUNK
ROWS_PER_TILE = N_PAD // 16

_MESH = plsc.VectorSubcoreMesh(core_axis_name="c", subcore_axis_name="s")

_E_SPLIT = 16 * G_C0 * CHUNK


def _edge_layout(flat, pad_value):
    a = flat[:_E_SPLIT].reshape(16, G_C0, CHUNK)
    b = flat[_E_SPLIT:].reshape(16, G_C1, CHUNK)
    if G_C0 > G_C1:
        b = jnp.pad(b, ((0, 0), (0, GMAX - G_C1), (0, 0)), constant_values=pad_value)
    else:
        a = jnp.pad(a, ((0, 0), (0, GMAX - G_C0), (0, 0)), constant_values=pad_value)
    return jnp.concatenate([a, b], axis=0)


@functools.partial(
    pl.kernel,
    out_type=jax.ShapeDtypeStruct((NTILES, N_PAD), jnp.float32),
    mesh=_MESH,
    compiler_params=pltpu.CompilerParams(needs_layout_passes=False),
    scratch_types=[
        pltpu.VMEM((GMAX, CHUNK), jnp.int32),
        pltpu.VMEM((N_PAD,), jnp.float32),
    ],
)
def _deg_kernel(col_hbm, zeros1d_hbm, out_hbm, col_vm, acc_vm):
    c = lax.axis_index("c")
    s = lax.axis_index("s")
    w = c * 16 + s
    pltpu.sync_copy(zeros1d_hbm, acc_vm)
    pltpu.sync_copy(col_hbm.at[w], col_vm)
    ones = jnp.full((16,), 1.0, dtype=jnp.float32)

    def body(g, carry):
        for j in range(CHUNK // 16):
            idx = col_vm[g, pl.ds(j * 16, 16)]
            plsc.addupdate_scatter(acc_vm, [idx], ones)
        return carry

    @pl.when(c == 0)
    def _():
        lax.fori_loop(0, G_C0, body, 0)

    @pl.when(c != 0)
    def _():
        lax.fori_loop(0, G_C1, body, 0)

    pltpu.sync_copy(acc_vm, out_hbm.at[w])


@functools.partial(
    pl.kernel,
    out_type=jax.ShapeDtypeStruct((2, N_PAD, 128), jnp.float32),
    mesh=_MESH,
    scratch_types=[
        pltpu.VMEM((GMAX, CHUNK), jnp.int32),
        pltpu.VMEM((GMAX, CHUNK), jnp.int32),
        pltpu.VMEM((CHUNK, 128), jnp.float32),
        pltpu.SemaphoreType.DMA,
        pltpu.VMEM_SHARED((N_PAD, 128), jnp.float32),
    ],
)
def _msg_kernel(hs_hbm, row_hbm, col_hbm, zeros2d_hbm, out_hbm,
                row_vm, col_vm, buf, sem, acc_sh):
    c = lax.axis_index("c")
    s = lax.axis_index("s")
    w = c * 16 + s
    pltpu.sync_copy(zeros2d_hbm, acc_sh.at[pl.ds(s * ROWS_PER_TILE, ROWS_PER_TILE)])
    pltpu.sync_copy(row_hbm.at[w], row_vm)
    pltpu.sync_copy(col_hbm.at[w], col_vm)
    plsc.subcore_barrier()

    def body(g, carry):
        pltpu.async_copy(hs_hbm.at[row_vm.at[g]], buf, sem).wait()
        pltpu.sync_copy(buf, acc_sh.at[col_vm.at[g]], add=True)
        return carry

    @pl.when(c == 0)
    def _():
        lax.fori_loop(0, G_C0, body, 0)

    @pl.when(c != 0)
    def _():
        lax.fori_loop(0, G_C1, body, 0)

    plsc.subcore_barrier()
    pltpu.sync_copy(
        acc_sh.at[pl.ds(s * ROWS_PER_TILE, ROWS_PER_TILE)],
        out_hbm.at[c, pl.ds(s * ROWS_PER_TILE, ROWS_PER_TILE)],
    )


_BLK = 2048
_GRID = N_PAD // _BLK
_DOT = dict(preferred_element_type=jnp.float32, precision=jax.lax.Precision.HIGHEST)


def _pre_body(xcat_ref, degp_ref, wblk_ref, bcat_ref, h0_ref, hs_ref, dis_ref):
    deg = jnp.sum(degp_ref[...], axis=0)
    dis = jnp.where(deg > 0, lax.rsqrt(jnp.maximum(deg, 1e-12)), 0.0)
    h = jnp.maximum(jnp.dot(xcat_ref[...], wblk_ref[...], **_DOT) + bcat_ref[...], 0.0)
    h0_ref[...] = h
    hs_ref[...] = h * dis[:, None]
    dis_ref[...] = jnp.broadcast_to(dis[:, None], (_BLK, 128))


_pre_call = pl.pallas_call(
    _pre_body,
    grid=(_GRID,),
    in_specs=[
        pl.BlockSpec((_BLK, 192), lambda i: (i, 0)),
        pl.BlockSpec((NTILES, _BLK), lambda i: (0, i)),
        pl.BlockSpec((192, 128), lambda i: (0, 0)),
        pl.BlockSpec((1, 128), lambda i: (0, 0)),
    ],
    out_specs=[
        pl.BlockSpec((_BLK, 128), lambda i: (i, 0)),
        pl.BlockSpec((_BLK, 128), lambda i: (i, 0)),
        pl.BlockSpec((_BLK, 128), lambda i: (i, 0)),
    ],
    out_shape=[jax.ShapeDtypeStruct((N_PAD, 128), jnp.float32)] * 3,
)


def _layer_body(agg_ref, h0_ref, dis_ref, w1_ref, w2_ref, out_ref, *, beta):
    agg = (agg_ref[0] + agg_ref[1]) * dis_ref[...] * (1.0 - ALPHA)
    h0a = h0_ref[...] * ALPHA
    t = (1.0 - beta) * (agg + h0a) + beta * (
        jnp.dot(agg, w1_ref[...], **_DOT) + jnp.dot(h0a, w2_ref[...], **_DOT))
    out_ref[...] = jnp.maximum(t, 0.0) * dis_ref[...]


def _final_body(agg_ref, h0_ref, dis_ref, w1_ref, w2_ref, wz_ref, bz_ref,
                out_ref, *, beta):
    agg = (agg_ref[0] + agg_ref[1]) * dis_ref[...] * (1.0 - ALPHA)
    h0a = h0_ref[...] * ALPHA
    t = (1.0 - beta) * (agg + h0a) + beta * (
        jnp.dot(agg, w1_ref[...], **_DOT) + jnp.dot(h0a, w2_ref[...], **_DOT))
    h = jnp.maximum(t, 0.0)
    out_ref[...] = jnp.dot(h, wz_ref[...], **_DOT) + bz_ref[...]


def _make_layer_call(body, beta, extra_specs=(), n_extra=0):
    return pl.pallas_call(
        functools.partial(body, beta=beta),
        grid=(_GRID,),
        in_specs=[
            pl.BlockSpec((2, _BLK, 128), lambda i: (0, i, 0)),
            pl.BlockSpec((_BLK, 128), lambda i: (i, 0)),
            pl.BlockSpec((_BLK, 128), lambda i: (i, 0)),
            pl.BlockSpec((128, 128), lambda i: (0, 0)),
            pl.BlockSpec((128, 128), lambda i: (0, 0)),
            *extra_specs,
        ],
        out_specs=pl.BlockSpec((_BLK, 128), lambda i: (i, 0)),
        out_shape=jax.ShapeDtypeStruct((N_PAD, 128), jnp.float32),
    )


def _beta(i):
    return float(math.log(THETA / (i + 1) + 1.0))


_layer1_call = _make_layer_call(_layer_body, _beta(0))
_final_call = _make_layer_call(
    _final_body, _beta(1),
    extra_specs=(pl.BlockSpec((128, 128), lambda i: (0, 0)),
                 pl.BlockSpec((1, 128), lambda i: (0, 0))),
)


def kernel(x, data_str, edge_index, lins0_w, lins0_b, lins1_w, lins1_b,
           lin11_w, lin11_b, lin3_w, lin3_b,
           convs_w1, convs_w2, convs1_w1, convs1_w2):
    f32 = jnp.float32
    row = edge_index[0]
    col = edge_index[1]
    row_p = jnp.concatenate([row, jnp.zeros((E_CAP - E,), jnp.int32)])
    col_p = jnp.concatenate([col, jnp.full((E_CAP - E,), N, jnp.int32)])
    row_t = _edge_layout(row_p, 0)
    col_t = _edge_layout(col_p, N)

    x_p = jnp.pad(x, ((0, N_PAD - N), (0, 0)))
    ds_p = jnp.pad(data_str, ((0, N_PAD - N), (0, 64 - D_STR)))
    xcat = jnp.concatenate([x_p, ds_p], axis=1)

    zero64 = jnp.zeros((64, 64), f32)
    wblk = jnp.concatenate([
        jnp.concatenate([lins0_w, jnp.zeros((128, 64), f32)], axis=1),
        jnp.concatenate([jnp.zeros((64, 64), f32),
                         jnp.pad(lin11_w, ((0, 64 - D_STR), (0, 0)))], axis=1),
    ], axis=0)
    bcat = jnp.concatenate([lins0_b, lin11_b]).reshape(1, 128)

    def blkdiag(a, b):
        return jnp.concatenate([
            jnp.concatenate([a, zero64], axis=1),
            jnp.concatenate([zero64, b], axis=1),
        ], axis=0)

    w1b = [blkdiag(convs_w1[i], convs1_w1[i]) for i in range(2)]
    w2b = [blkdiag(convs_w2[i], convs1_w2[i]) for i in range(2)]

    wz = jnp.zeros((128, 128), f32)
    wz = wz.at[:64, 0:1].set(lins1_w)
    wz = wz.at[64:, 1:2].set(lin3_w)
    bz = jnp.zeros((1, 128), f32)
    bz = bz.at[0, 0].set(lins1_b[0])
    bz = bz.at[0, 1].set(lin3_b[0])

    zeros1d = jnp.zeros((N_PAD,), f32)
    zeros2d = jnp.zeros((ROWS_PER_TILE, 128), f32)

    degp = _deg_kernel(col_t, zeros1d)
    h0, hs, dis = _pre_call(xcat, degp, wblk, bcat)

    agg1 = _msg_kernel(hs, row_t, col_t, zeros2d)
    hs1 = _layer1_call(agg1, h0, dis, w1b[0], w2b[0])

    agg2 = _msg_kernel(hs1, row_t, col_t, zeros2d)
    zcat = _final_call(agg2, h0, dis, w1b[1], w2b[1], wz, bz)

    return (zcat[:N, 0:1], zcat[:N, 1:2])
